# Initial kernel scaffold; baseline (speedup 1.0000x reference)
import jax
import jax.numpy as jnp
from jax.experimental import pallas as pl


def kernel(node_features, edge_index, W_in, b_in, W_lp1, b_lp1, W_lp2, b_lp2, Wq, bq, Wk, bk, Wv, bv, Wo, bo, Wg, bg, W1, b1, W2, b2, g1, be1, g2, be2, g3, be3, gf, bf, W_out, b_out):
    raise NotImplementedError("write your pallas kernel here")


# speedup vs baseline: 2.9920x; 2.9920x over previous
import functools

import jax
import jax.numpy as jnp
from jax import lax
from jax.experimental import pallas as pl
from jax.experimental.pallas import tpu as pltpu

N = 10000
E = 160000
HID = 512
HEADS = 8
HD = HID // HEADS
NCHUNK = 4
CW = HID // NCHUNK

BR = 400
GR = N // BR
BE = 2000
GE = E // BE

_f32 = jnp.float32


def _bs(shape, imap):
    return pl.BlockSpec(shape, imap)


def _const(shape):
    return pl.BlockSpec(shape, lambda i: (0,) * len(shape))


def _t0_body(x, Win, bin_, Wt, Wb, blp, h_o, a_o, b_o):
    h = jnp.dot(x[...], Win[...], preferred_element_type=_f32) + bin_[...]
    h_o[...] = h
    a_o[...] = jnp.dot(h, Wt[...], preferred_element_type=_f32) + blp[...]
    b_o[...] = jnp.dot(h, Wb[...], preferred_element_type=_f32)


def _t0(x, Win, bin_, Wt, Wb, blp):
    d_in = x.shape[1]
    return pl.pallas_call(
        _t0_body, grid=(GR,),
        in_specs=[_bs((BR, d_in), lambda i: (i, 0)), _const((d_in, HID)),
                  _const((1, HID)), _const((HID, HID)), _const((HID, HID)),
                  _const((1, HID))],
        out_specs=[_bs((BR, HID), lambda i: (i, 0))] * 3,
        out_shape=[jax.ShapeDtypeStruct((N, HID), _f32)] * 3,
    )(x, Win, bin_, Wt, Wb, blp)


def _qkv_body(h, Wq, bq, Wk, bk, Wv, bv, q_o, k_o, v_o):
    hh = h[...]
    q_o[...] = jnp.dot(hh, Wq[...], preferred_element_type=_f32) + bq[...]
    k_o[...] = jnp.dot(hh, Wk[...], preferred_element_type=_f32) + bk[...]
    v_o[...] = jnp.dot(hh, Wv[...], preferred_element_type=_f32) + bv[...]


def _qkv(h, Wq, bq, Wk, bk, Wv, bv):
    return pl.pallas_call(
        _qkv_body, grid=(GR,),
        in_specs=[_bs((BR, HID), lambda i: (i, 0))] +
                 [_const((HID, HID)), _const((1, HID))] * 3,
        out_specs=[_bs((BR, HID), lambda i: (i, 0))] * 3,
        out_shape=[jax.ShapeDtypeStruct((N, HID), _f32)] * 3,
    )(h, Wq, bq, Wk, bk, Wv, bv)


def _gmax_body(sc, o):
    i = pl.program_id(0)

    @pl.when(i == 0)
    def _():
        o[...] = jnp.full_like(o[...], -jnp.inf)

    o[...] = jnp.maximum(o[...], jnp.max(sc[...], axis=0, keepdims=True))


def _gmax(sc):
    return pl.pallas_call(
        _gmax_body, grid=(GE,),
        in_specs=[_bs((BE, HEADS), lambda i: (i, 0))],
        out_specs=_bs((1, HEADS), lambda i: (0, 0)),
        out_shape=jax.ShapeDtypeStruct((1, HEADS), _f32),
    )(sc)


def _exp_body(sc, gm, o):
    o[...] = jnp.exp(sc[...] - gm[...])


def _expk(sc, gm):
    return pl.pallas_call(
        _exp_body, grid=(GE,),
        in_specs=[_bs((BE, HEADS), lambda i: (i, 0)), _const((1, HEADS))],
        out_specs=_bs((BE, HEADS), lambda i: (i, 0)),
        out_shape=jax.ShapeDtypeStruct((E, HEADS), _f32),
    )(sc, gm)


def _ln(x, g, b, eps=1e-5):
    mu = jnp.mean(x, axis=-1, keepdims=True)
    var = jnp.mean((x - mu) ** 2, axis=-1, keepdims=True)
    return (x - mu) * lax.rsqrt(var + eps) * g + b


def _attn_body(h, u0, u1, u2, u3, dnm, dgo, erep, Wo, bo, g1, be1,
               h_o, f_o):
    u = jnp.concatenate(
        [u0[0] + u0[1], u1[0] + u1[1], u2[0] + u2[1], u3[0] + u3[1]], axis=1)
    den = dnm[0] + dnm[1]
    denx = jnp.dot(den, erep[...], preferred_element_type=_f32)
    hag = u / jnp.maximum(denx, 1e-30)
    h_attn = jnp.dot(hag, Wo[...], preferred_element_type=_f32) + bo[...]
    y = _ln(h[...] + h_attn, g1[...], be1[...])
    h_o[...] = y
    deg = jnp.maximum(dgo[0][:, :1] + dgo[1][:, :1], 1.0)
    f_o[...] = y * lax.rsqrt(deg)


def _attn(h, parts, dnm, dgo, erep, Wo, bo, g1, be1):
    return pl.pallas_call(
        _attn_body, grid=(GR,),
        in_specs=[_bs((BR, HID), lambda i: (i, 0))] +
                 [_bs((2, BR, CW), lambda i: (0, i, 0))] * 4 +
                 [_bs((2, BR, HEADS), lambda i: (0, i, 0))] * 2 +
                 [_const((HEADS, HID)), _const((HID, HID)), _const((1, HID)),
                  _const((1, HID)), _const((1, HID))],
        out_specs=[_bs((BR, HID), lambda i: (i, 0))] * 2,
        out_shape=[jax.ShapeDtypeStruct((N, HID), _f32)] * 2,
    )(h, *parts, dnm, dgo, erep, Wo, bo, g1, be1)


def _gcff_body(h, a0, a1, a2, a3, dgi, Wg, bg, g2, be2, W1, b1, W2, b2,
               g3, be3, h_o):
    agg = jnp.concatenate(
        [a0[0] + a0[1], a1[0] + a1[1], a2[0] + a2[1], a3[0] + a3[1]], axis=1)
    deg = jnp.maximum(dgi[0][:, :1] + dgi[1][:, :1], 1.0)
    agg = agg * lax.rsqrt(deg)
    hs = jnp.dot(agg, Wg[...], preferred_element_type=_f32) + bg[...]
    y = _ln(h[...] + hs, g2[...], be2[...])
    f = jnp.dot(jax.nn.relu(
        jnp.dot(y, W1[...], preferred_element_type=_f32) + b1[...]),
        W2[...], preferred_element_type=_f32) + b2[...]
    h_o[...] = _ln(y + f, g3[...], be3[...])


def _gcff(h, parts, dgi, Wg, bg, g2, be2, W1, b1, W2, b2, g3, be3):
    dff = W1.shape[1]
    return pl.pallas_call(
        _gcff_body, grid=(GR,),
        in_specs=[_bs((BR, HID), lambda i: (i, 0))] +
                 [_bs((2, BR, CW), lambda i: (0, i, 0))] * 4 +
                 [_bs((2, BR, HEADS), lambda i: (0, i, 0))] +
                 [_const((HID, HID)), _const((1, HID)), _const((1, HID)),
                  _const((1, HID)), _const((HID, dff)), _const((1, dff)),
                  _const((dff, HID)), _const((1, HID)), _const((1, HID)),
                  _const((1, HID))],
        out_specs=_bs((BR, HID), lambda i: (i, 0)),
        out_shape=jax.ShapeDtypeStruct((N, HID), _f32),
    )(h, *parts, dgi, Wg, bg, g2, be2, W1, b1, W2, b2, g3, be3)


def _final_body(h, gf, bf, Wout, bout, o):
    y = _ln(h[...], gf[...], bf[...])
    o[...] = jnp.dot(y, Wout[...], preferred_element_type=_f32) + bout[...]


def _final(h, gf, bf, Wout, bout):
    dout = Wout.shape[1]
    return pl.pallas_call(
        _final_body, grid=(GR,),
        in_specs=[_bs((BR, HID), lambda i: (i, 0)), _const((1, HID)),
                  _const((1, HID)), _const((HID, dout)), _const((1, dout))],
        out_specs=_bs((BR, dout), lambda i: (i, 0)),
        out_shape=jax.ShapeDtypeStruct((N, dout), _f32),
    )(h, gf, bf, Wout, bout)


def _edge_lp(A2, B2, w2, b2s, src, dst):
    r = jax.nn.relu(A2[src] + B2[dst])
    s = r @ w2 + b2s
    return jax.nn.sigmoid(s)


def _edge_scores(q, k, src, dst):
    qs = q[src].reshape(E, HEADS, HD)
    ks = k[dst].reshape(E, HEADS, HD)
    return jnp.sum(qs * ks, axis=-1) * (HD ** -0.5)


def _edge_scatter(tables, scales, src, dst):
    outs = []
    for c, tab in enumerate(tables):
        rows = tab[src]
        if scales.shape[1] == HEADS:
            sc = jnp.repeat(scales[:, 2 * c:2 * c + 2], HD, axis=1)
        else:
            sc = scales
        full = jax.ops.segment_sum(rows * sc, dst, num_segments=N)
        outs.append(jnp.stack([full, jnp.zeros_like(full)]))
    return outs


def _scatter_small(vals, idx):
    full = jax.ops.segment_sum(vals, idx, num_segments=N)
    return jnp.stack([full, jnp.zeros_like(full)])


def kernel(node_features, edge_index, W_in, b_in, W_lp1, b_lp1, W_lp2, b_lp2,
           Wq, bq, Wk, bk, Wv, bv, Wo, bo, Wg, bg, W1, b1, W2, b2,
           g1, be1, g2, be2, g3, be3, gf, bf, W_out, b_out):
    src = edge_index[0]
    dst = edge_index[1]
    r2 = lambda v: v.reshape(1, -1)

    h, A2, B2 = _t0(node_features, W_in, r2(b_in), W_lp1[:HID], W_lp1[HID:],
                    r2(b_lp1))
    ew = _edge_lp(A2, B2, W_lp2[:, 0], b_lp2[0], src, dst)

    ones = jnp.ones((E, HEADS), _f32)
    dgo = _scatter_small(ones, src)
    dgi = _scatter_small(ones, dst)

    erep = jnp.repeat(jnp.eye(HEADS, dtype=_f32), HD, axis=1)

    for l in range(3):
        q, k, v = _qkv(h, Wq[l], r2(bq[l]), Wk[l], r2(bk[l]), Wv[l],
                       r2(bv[l]))
        sc = _edge_scores(q, k, src, dst)
        e = _expk(sc, _gmax(sc))
        dnm = _scatter_small(e, dst)
        vtabs = [v[:, c * CW:(c + 1) * CW] for c in range(NCHUNK)]
        parts = _edge_scatter(vtabs, e, src, dst)
        h, feat = _attn(h, parts, dnm, dgo, erep, Wo[l], r2(bo[l]),
                        r2(g1[l]), r2(be1[l]))
        ftabs = [feat[:, c * CW:(c + 1) * CW] for c in range(NCHUNK)]
        gparts = _edge_scatter(ftabs, ew[:, None], src, dst)
        h = _gcff(h, gparts, dgi, Wg[l], r2(bg[l]), r2(g2[l]), r2(be2[l]),
                  W1[l], r2(b1[l]), W2[l], r2(b2[l]), r2(g3[l]), r2(be3[l]))

    return _final(h, r2(gf), r2(bf), W_out, r2(b_out))


# speedup vs baseline: 4.8311x; 1.6147x over previous
import functools

import jax
import jax.numpy as jnp
from jax import lax
from jax.experimental import pallas as pl
from jax.experimental.pallas import tpu as pltpu
from jax.experimental.pallas import tpu_sc as plsc

N = 10000
E = 160000
HID = 512
HEADS = 8
HD = HID // HEADS
NCHUNK = 4
CW = HID // NCHUNK

BR = 400
GR = N // BR
BE = 2000
GE = E // BE

_f32 = jnp.float32


def _bs(shape, imap):
    return pl.BlockSpec(shape, imap)


def _const(shape):
    return pl.BlockSpec(shape, lambda i: (0,) * len(shape))


def _t0_body(x, Win, bin_, Wt, Wb, blp, h_o, a_o, b_o):
    h = jnp.dot(x[...], Win[...], preferred_element_type=_f32) + bin_[...]
    h_o[...] = h
    a_o[...] = jnp.dot(h, Wt[...], preferred_element_type=_f32) + blp[...]
    b_o[...] = jnp.dot(h, Wb[...], preferred_element_type=_f32)


def _t0(x, Win, bin_, Wt, Wb, blp):
    d_in = x.shape[1]
    return pl.pallas_call(
        _t0_body, grid=(GR,),
        in_specs=[_bs((BR, d_in), lambda i: (i, 0)), _const((d_in, HID)),
                  _const((1, HID)), _const((HID, HID)), _const((HID, HID)),
                  _const((1, HID))],
        out_specs=[_bs((BR, HID), lambda i: (i, 0))] * 3,
        out_shape=[jax.ShapeDtypeStruct((N, HID), _f32)] * 3,
    )(x, Win, bin_, Wt, Wb, blp)


def _qkv_body(h, Wq, bq, Wk, bk, Wv, bv, q_o, k_o, v_o):
    hh = h[...]
    q_o[...] = jnp.dot(hh, Wq[...], preferred_element_type=_f32) + bq[...]
    k_o[...] = jnp.dot(hh, Wk[...], preferred_element_type=_f32) + bk[...]
    v_o[...] = jnp.dot(hh, Wv[...], preferred_element_type=_f32) + bv[...]


def _qkv(h, Wq, bq, Wk, bk, Wv, bv):
    return pl.pallas_call(
        _qkv_body, grid=(GR,),
        in_specs=[_bs((BR, HID), lambda i: (i, 0))] +
                 [_const((HID, HID)), _const((1, HID))] * 3,
        out_specs=[_bs((BR, HID), lambda i: (i, 0))] * 3,
        out_shape=[jax.ShapeDtypeStruct((N, HID), _f32)] * 3,
    )(h, Wq, bq, Wk, bk, Wv, bv)


def _gmax_body(sc, o):
    i = pl.program_id(0)

    @pl.when(i == 0)
    def _():
        o[...] = jnp.full_like(o[...], -jnp.inf)

    o[...] = jnp.maximum(o[...], jnp.max(sc[...], axis=0, keepdims=True))


def _gmax(sc):
    return pl.pallas_call(
        _gmax_body, grid=(GE,),
        in_specs=[_bs((BE, HEADS), lambda i: (i, 0))],
        out_specs=_bs((1, HEADS), lambda i: (0, 0)),
        out_shape=jax.ShapeDtypeStruct((1, HEADS), _f32),
    )(sc)


def _exp_body(sc, gm, o):
    o[...] = jnp.exp(sc[...] - gm[...])


def _expk(sc, gm):
    return pl.pallas_call(
        _exp_body, grid=(GE,),
        in_specs=[_bs((BE, HEADS), lambda i: (i, 0)), _const((1, HEADS))],
        out_specs=_bs((BE, HEADS), lambda i: (i, 0)),
        out_shape=jax.ShapeDtypeStruct((E, HEADS), _f32),
    )(sc, gm)


def _ln(x, g, b, eps=1e-5):
    mu = jnp.mean(x, axis=-1, keepdims=True)
    var = jnp.mean((x - mu) ** 2, axis=-1, keepdims=True)
    return (x - mu) * lax.rsqrt(var + eps) * g + b


def _attn_body(h, u0, u1, u2, u3, dnm, dgo, erep, Wo, bo, g1, be1,
               h_o, f_o):
    u = jnp.concatenate(
        [u0[0] + u0[1], u1[0] + u1[1], u2[0] + u2[1], u3[0] + u3[1]], axis=1)
    den = dnm[0][:, :HEADS] + dnm[1][:, :HEADS]
    denx = jnp.dot(den, erep[...], preferred_element_type=_f32)
    hag = u / jnp.maximum(denx, 1e-30)
    h_attn = jnp.dot(hag, Wo[...], preferred_element_type=_f32) + bo[...]
    y = _ln(h[...] + h_attn, g1[...], be1[...])
    h_o[...] = y
    deg = jnp.maximum(dgo[0][:, :1] + dgo[1][:, :1], 1.0)
    f_o[...] = y * lax.rsqrt(deg)


def _attn(h, parts, dnm, dgo, erep, Wo, bo, g1, be1):
    return pl.pallas_call(
        _attn_body, grid=(GR,),
        in_specs=[_bs((BR, HID), lambda i: (i, 0))] +
                 [_bs((2, BR, CW), lambda i: (0, i, 0))] * 6 +
                 [_const((HEADS, HID)), _const((HID, HID)), _const((1, HID)),
                  _const((1, HID)), _const((1, HID))],
        out_specs=[_bs((BR, HID), lambda i: (i, 0))] * 2,
        out_shape=[jax.ShapeDtypeStruct((N, HID), _f32)] * 2,
    )(h, *parts, dnm, dgo, erep, Wo, bo, g1, be1)


def _gcff_body(h, a0, a1, a2, a3, dgi, Wg, bg, g2, be2, W1, b1, W2, b2,
               g3, be3, h_o):
    agg = jnp.concatenate(
        [a0[0] + a0[1], a1[0] + a1[1], a2[0] + a2[1], a3[0] + a3[1]], axis=1)
    deg = jnp.maximum(dgi[0][:, :1] + dgi[1][:, :1], 1.0)
    agg = agg * lax.rsqrt(deg)
    hs = jnp.dot(agg, Wg[...], preferred_element_type=_f32) + bg[...]
    y = _ln(h[...] + hs, g2[...], be2[...])
    f = jnp.dot(jax.nn.relu(
        jnp.dot(y, W1[...], preferred_element_type=_f32) + b1[...]),
        W2[...], preferred_element_type=_f32) + b2[...]
    h_o[...] = _ln(y + f, g3[...], be3[...])


def _gcff(h, parts, dgi, Wg, bg, g2, be2, W1, b1, W2, b2, g3, be3):
    dff = W1.shape[1]
    return pl.pallas_call(
        _gcff_body, grid=(GR,),
        in_specs=[_bs((BR, HID), lambda i: (i, 0))] +
                 [_bs((2, BR, CW), lambda i: (0, i, 0))] * 5 +
                 [_const((HID, HID)), _const((1, HID)), _const((1, HID)),
                  _const((1, HID)), _const((HID, dff)), _const((1, dff)),
                  _const((dff, HID)), _const((1, HID)), _const((1, HID)),
                  _const((1, HID))],
        out_specs=_bs((BR, HID), lambda i: (i, 0)),
        out_shape=jax.ShapeDtypeStruct((N, HID), _f32),
    )(h, *parts, dgi, Wg, bg, g2, be2, W1, b1, W2, b2, g3, be3)


def _final_body(h, gf, bf, Wout, bout, o):
    y = _ln(h[...], gf[...], bf[...])
    o[...] = jnp.dot(y, Wout[...], preferred_element_type=_f32) + bout[...]


def _final(h, gf, bf, Wout, bout):
    dout = Wout.shape[1]
    return pl.pallas_call(
        _final_body, grid=(GR,),
        in_specs=[_bs((BR, HID), lambda i: (i, 0)), _const((1, HID)),
                  _const((1, HID)), _const((HID, dout)), _const((1, dout))],
        out_specs=_bs((BR, dout), lambda i: (i, 0)),
        out_shape=jax.ShapeDtypeStruct((N, dout), _f32),
    )(h, gf, bf, Wout, bout)


def _score_body(qs, ks, ebd, o):
    o[...] = jnp.dot(qs[...] * ks[...], ebd[...],
                     preferred_element_type=_f32) * (HD ** -0.5)


def _score_tc(gq, gk, ebd):
    return pl.pallas_call(
        _score_body, grid=(GE,),
        in_specs=[_bs((BE, HID), lambda i: (i, 0))] * 2 +
                 [_const((HID, HEADS))],
        out_specs=_bs((BE, HEADS), lambda i: (i, 0)),
        out_shape=jax.ShapeDtypeStruct((E, HEADS), _f32),
    )(gq, gk, ebd)


def _lp_body(a, b, w2p, b2s, o):
    r = jax.nn.relu(a[...] + b[...])
    o[...] = jax.nn.sigmoid(
        jnp.dot(r, w2p[...], preferred_element_type=_f32) + b2s[...])


def _lp_tc(ga, gb, w2p, b2s):
    return pl.pallas_call(
        _lp_body, grid=(GE,),
        in_specs=[_bs((BE, HID), lambda i: (i, 0))] * 2 +
                 [_const((HID, HEADS)), _const((1, HEADS))],
        out_specs=_bs((BE, HEADS), lambda i: (i, 0)),
        out_shape=jax.ShapeDtypeStruct((E, HEADS), _f32),
    )(ga, gb, w2p, b2s)


_INFO = plsc.get_sparse_core_info()
_NC = _INFO.num_cores
_NS = _INFO.num_subcores
_NW = _NC * _NS
EPW = E // _NW
CH = 40
NIT = EPW // CH
RB = 624
RTAIL = N - RB * _NS

_mesh = plsc.VectorSubcoreMesh(core_axis_name="c", subcore_axis_name="s")


def _make_gather(D):
    @functools.partial(
        pl.kernel, mesh=_mesh,
        out_type=jax.ShapeDtypeStruct((E, D), _f32),
        scratch_types=[pltpu.VMEM((CH,), jnp.int32),
                       pltpu.VMEM((CH, D), _f32),
                       pltpu.SemaphoreType.DMA])
    def k(tab, idx, out, idx_v, rows_v, sem):
        wid = lax.axis_index("s") * _NC + lax.axis_index("c")

        def body(i, _):
            base = wid * EPW + i * CH
            pltpu.sync_copy(idx.at[pl.ds(base, CH)], idx_v)
            pltpu.async_copy(tab.at[idx_v], rows_v, sem).wait()
            pltpu.sync_copy(rows_v, out.at[pl.ds(base, CH)])
            return 0

        lax.fori_loop(0, NIT, body, 0)

    return k


def _make_scatter_rows(W):
    @functools.partial(
        pl.kernel, mesh=_mesh,
        out_type=jax.ShapeDtypeStruct((_NC, N, W), _f32),
        scratch_types=[pltpu.VMEM((CH,), jnp.int32),
                       pltpu.VMEM((CH, W), _f32),
                       pltpu.VMEM_SHARED((N, W), _f32)])
    def k(vals, idx, z, out, idx_v, vals_v, acc):
        cid = lax.axis_index("c")
        sid = lax.axis_index("s")
        wid = sid * _NC + cid
        r0 = sid * RB
        pltpu.sync_copy(z.at[pl.ds(r0, RB)], acc.at[pl.ds(r0, RB)])

        @pl.when(sid == _NS - 1)
        def _():
            pltpu.sync_copy(z.at[pl.ds(RB * _NS, RTAIL)],
                            acc.at[pl.ds(RB * _NS, RTAIL)])

        plsc.subcore_barrier()

        def body(i, _):
            base = wid * EPW + i * CH
            pltpu.sync_copy(idx.at[pl.ds(base, CH)], idx_v)
            pltpu.sync_copy(vals.at[pl.ds(base, CH)], vals_v)
            pltpu.sync_copy(vals_v, acc.at[idx_v], add=True)
            return 0

        lax.fori_loop(0, NIT, body, 0)
        plsc.subcore_barrier()
        pltpu.sync_copy(acc.at[pl.ds(r0, RB)], out.at[cid, pl.ds(r0, RB)])

        @pl.when(sid == _NS - 1)
        def _():
            pltpu.sync_copy(acc.at[pl.ds(RB * _NS, RTAIL)],
                            out.at[cid, pl.ds(RB * _NS, RTAIL)])

    return k


def _make_gss(S):
    @functools.partial(
        pl.kernel, mesh=_mesh,
        out_type=jax.ShapeDtypeStruct((_NC, N, CW), _f32),
        scratch_types=[pltpu.VMEM((CH,), jnp.int32),
                       pltpu.VMEM((CH,), jnp.int32),
                       pltpu.VMEM((CH * S + 16,), _f32),
                       pltpu.VMEM((CH, CW), _f32),
                       pltpu.VMEM_SHARED((N, CW), _f32),
                       pltpu.SemaphoreType.DMA])
    def k(tab, scl, gidx, sidx, z, out, gi_v, si_v, s_v, rows_v, acc, sem):
        cid = lax.axis_index("c")
        sid = lax.axis_index("s")
        wid = sid * _NC + cid
        r0 = sid * RB
        pltpu.sync_copy(z.at[pl.ds(r0, RB)], acc.at[pl.ds(r0, RB)])

        @pl.when(sid == _NS - 1)
        def _():
            pltpu.sync_copy(z.at[pl.ds(RB * _NS, RTAIL)],
                            acc.at[pl.ds(RB * _NS, RTAIL)])

        plsc.subcore_barrier()

        def body(i, _):
            base = wid * EPW + i * CH
            pltpu.sync_copy(gidx.at[pl.ds(base, CH)], gi_v)
            pltpu.sync_copy(sidx.at[pl.ds(base, CH)], si_v)
            pltpu.sync_copy(scl.at[pl.ds(base * S, CH * S)],
                            s_v.at[pl.ds(0, CH * S)])
            pltpu.async_copy(tab.at[gi_v], rows_v, sem).wait()

            def scale_edge(j, _):
                s0 = s_v[pl.ds(j * S, 16)][0]
                s1 = s0 if S == 1 else s_v[pl.ds(j * S + 1, 16)][0]
                for t in range(CW // 16):
                    s = s0 if t < (CW // 32) else s1
                    rows_v[j, pl.ds(t * 16, 16)] = (
                        rows_v[j, pl.ds(t * 16, 16)] * s)
                return 0

            lax.fori_loop(0, CH, scale_edge, 0)
            pltpu.sync_copy(rows_v, acc.at[si_v], add=True)
            return 0

        lax.fori_loop(0, NIT, body, 0)
        plsc.subcore_barrier()
        pltpu.sync_copy(acc.at[pl.ds(r0, RB)], out.at[cid, pl.ds(r0, RB)])

        @pl.when(sid == _NS - 1)
        def _():
            pltpu.sync_copy(acc.at[pl.ds(RB * _NS, RTAIL)],
                            out.at[cid, pl.ds(RB * _NS, RTAIL)])

    return k


_GATHER512 = _make_gather(HID)
_SCATC = _make_scatter_rows(CW)
_GSS1 = _make_gss(1)
_GSS2 = _make_gss(2)


def kernel(node_features, edge_index, W_in, b_in, W_lp1, b_lp1, W_lp2, b_lp2,
           Wq, bq, Wk, bk, Wv, bv, Wo, bo, Wg, bg, W1, b1, W2, b2,
           g1, be1, g2, be2, g3, be3, gf, bf, W_out, b_out):
    src = edge_index[0]
    dst = edge_index[1]
    r2 = lambda v: v.reshape(1, -1)

    h, A2, B2 = _t0(node_features, W_in, r2(b_in), W_lp1[:HID], W_lp1[HID:],
                    r2(b_lp1))
    ga = _GATHER512(A2, src)
    gb = _GATHER512(B2, dst)
    w2p = jnp.tile(W_lp2, (1, HEADS))
    b2s = jnp.broadcast_to(b_lp2.reshape(1, 1), (1, HEADS))
    ews = _lp_tc(ga, gb, w2p, b2s)[:, 0]

    ones = jnp.ones((E, CW), _f32)
    z128 = jnp.zeros((N, CW), _f32)
    epad = jnp.zeros((E, CW - HEADS), _f32)
    dgo = _SCATC(ones, src, z128)
    dgi = _SCATC(ones, dst, z128)

    erep = jnp.repeat(jnp.eye(HEADS, dtype=_f32), HD, axis=1)
    ebd = jnp.repeat(jnp.eye(HEADS, dtype=_f32), HD, axis=0)

    for l in range(3):
        q, k, v = _qkv(h, Wq[l], r2(bq[l]), Wk[l], r2(bk[l]), Wv[l],
                       r2(bv[l]))
        gq = _GATHER512(q, src)
        gk = _GATHER512(k, dst)
        sc = _score_tc(gq, gk, ebd)
        e = _expk(sc, _gmax(sc))
        dnm = _SCATC(jnp.concatenate([e, epad], axis=1), dst, z128)
        parts = [_GSS2(v[:, c * CW:(c + 1) * CW],
                       e[:, 2 * c:2 * c + 2].reshape(-1), src, dst, z128)
                 for c in range(NCHUNK)]
        h, feat = _attn(h, parts, dnm, dgo, erep, Wo[l], r2(bo[l]),
                        r2(g1[l]), r2(be1[l]))
        gparts = [_GSS1(feat[:, c * CW:(c + 1) * CW], ews, src, dst, z128)
                  for c in range(NCHUNK)]
        h = _gcff(h, gparts, dgi, Wg[l], r2(bg[l]), r2(g2[l]), r2(be2[l]),
                  W1[l], r2(b1[l]), W2[l], r2(b2[l]), r2(g3[l]), r2(be3[l]))

    return _final(h, r2(gf), r2(bf), W_out, r2(b_out))


# speedup vs baseline: 6.4151x; 1.3279x over previous
import functools

import jax
import jax.numpy as jnp
from jax import lax
from jax.experimental import pallas as pl
from jax.experimental.pallas import tpu as pltpu
from jax.experimental.pallas import tpu_sc as plsc

N = 10000
E = 160000
HID = 512
HEADS = 8
HD = HID // HEADS
NCHUNK = 4
CW = HID // NCHUNK

BR = 400
GR = N // BR
BE = 2000
GE = E // BE

_f32 = jnp.float32


def _bs(shape, imap):
    return pl.BlockSpec(shape, imap)


def _const(shape):
    return pl.BlockSpec(shape, lambda i: (0,) * len(shape))


def _t0_body(x, Win, bin_, Wt, Wb, blp, h_o, a_o, b_o):
    h = jnp.dot(x[...], Win[...], preferred_element_type=_f32) + bin_[...]
    h_o[...] = h
    a_o[...] = jnp.dot(h, Wt[...], preferred_element_type=_f32) + blp[...]
    b_o[...] = jnp.dot(h, Wb[...], preferred_element_type=_f32)


def _t0(x, Win, bin_, Wt, Wb, blp):
    d_in = x.shape[1]
    return pl.pallas_call(
        _t0_body, grid=(GR,),
        in_specs=[_bs((BR, d_in), lambda i: (i, 0)), _const((d_in, HID)),
                  _const((1, HID)), _const((HID, HID)), _const((HID, HID)),
                  _const((1, HID))],
        out_specs=[_bs((BR, HID), lambda i: (i, 0))] * 3,
        out_shape=[jax.ShapeDtypeStruct((N, HID), _f32)] * 3,
    )(x, Win, bin_, Wt, Wb, blp)


def _qkv_body(h, Wq, bq, Wk, bk, Wv, bv, q_o, k_o, v_o):
    hh = h[...]
    q_o[...] = jnp.dot(hh, Wq[...], preferred_element_type=_f32) + bq[...]
    k_o[...] = jnp.dot(hh, Wk[...], preferred_element_type=_f32) + bk[...]
    v_o[...] = jnp.dot(hh, Wv[...], preferred_element_type=_f32) + bv[...]


def _qkv(h, Wq, bq, Wk, bk, Wv, bv):
    return pl.pallas_call(
        _qkv_body, grid=(GR,),
        in_specs=[_bs((BR, HID), lambda i: (i, 0))] +
                 [_const((HID, HID)), _const((1, HID))] * 3,
        out_specs=[_bs((BR, HID), lambda i: (i, 0))] * 3,
        out_shape=[jax.ShapeDtypeStruct((N, HID), _f32)] * 3,
    )(h, Wq, bq, Wk, bk, Wv, bv)


def _gmax_body(sc, o):
    i = pl.program_id(0)

    @pl.when(i == 0)
    def _():
        o[...] = jnp.full_like(o[...], -jnp.inf)

    o[...] = jnp.maximum(o[...], jnp.max(sc[...], axis=0, keepdims=True))


def _gmax(sc):
    return pl.pallas_call(
        _gmax_body, grid=(GE,),
        in_specs=[_bs((BE, HEADS), lambda i: (i, 0))],
        out_specs=_bs((1, HEADS), lambda i: (0, 0)),
        out_shape=jax.ShapeDtypeStruct((1, HEADS), _f32),
    )(sc)


def _exp_body(sc, gm, o):
    o[...] = jnp.exp(sc[...] - gm[...])


def _expk(sc, gm):
    return pl.pallas_call(
        _exp_body, grid=(GE,),
        in_specs=[_bs((BE, HEADS), lambda i: (i, 0)), _const((1, HEADS))],
        out_specs=_bs((BE, HEADS), lambda i: (i, 0)),
        out_shape=jax.ShapeDtypeStruct((E, HEADS), _f32),
    )(sc, gm)


def _ln(x, g, b, eps=1e-5):
    mu = jnp.mean(x, axis=-1, keepdims=True)
    var = jnp.mean((x - mu) ** 2, axis=-1, keepdims=True)
    return (x - mu) * lax.rsqrt(var + eps) * g + b


def _attn_body(h, u0, u1, u2, u3, dnm, dgo, erep, Wo, bo, g1, be1,
               h_o, f_o):
    u = jnp.concatenate(
        [u0[0] + u0[1], u1[0] + u1[1], u2[0] + u2[1], u3[0] + u3[1]], axis=1)
    den = dnm[0][:, :HEADS] + dnm[1][:, :HEADS]
    denx = jnp.dot(den, erep[...], preferred_element_type=_f32)
    hag = u / jnp.maximum(denx, 1e-30)
    h_attn = jnp.dot(hag, Wo[...], preferred_element_type=_f32) + bo[...]
    y = _ln(h[...] + h_attn, g1[...], be1[...])
    h_o[...] = y
    deg = jnp.maximum(dgo[0][:, :1] + dgo[1][:, :1], 1.0)
    f_o[...] = y * lax.rsqrt(deg)


def _attn(h, parts, dnm, dgo, erep, Wo, bo, g1, be1):
    return pl.pallas_call(
        _attn_body, grid=(GR,),
        in_specs=[_bs((BR, HID), lambda i: (i, 0))] +
                 [_bs((2, BR, CW), lambda i: (0, i, 0))] * 6 +
                 [_const((HEADS, HID)), _const((HID, HID)), _const((1, HID)),
                  _const((1, HID)), _const((1, HID))],
        out_specs=[_bs((BR, HID), lambda i: (i, 0))] * 2,
        out_shape=[jax.ShapeDtypeStruct((N, HID), _f32)] * 2,
    )(h, *parts, dnm, dgo, erep, Wo, bo, g1, be1)


def _gcff_body(h, a0, a1, a2, a3, dgi, Wg, bg, g2, be2, W1, b1, W2, b2,
               g3, be3, h_o):
    agg = jnp.concatenate(
        [a0[0] + a0[1], a1[0] + a1[1], a2[0] + a2[1], a3[0] + a3[1]], axis=1)
    deg = jnp.maximum(dgi[0][:, :1] + dgi[1][:, :1], 1.0)
    agg = agg * lax.rsqrt(deg)
    hs = jnp.dot(agg, Wg[...], preferred_element_type=_f32) + bg[...]
    y = _ln(h[...] + hs, g2[...], be2[...])
    f = jnp.dot(jax.nn.relu(
        jnp.dot(y, W1[...], preferred_element_type=_f32) + b1[...]),
        W2[...], preferred_element_type=_f32) + b2[...]
    h_o[...] = _ln(y + f, g3[...], be3[...])


def _gcff(h, parts, dgi, Wg, bg, g2, be2, W1, b1, W2, b2, g3, be3):
    dff = W1.shape[1]
    return pl.pallas_call(
        _gcff_body, grid=(GR,),
        in_specs=[_bs((BR, HID), lambda i: (i, 0))] +
                 [_bs((2, BR, CW), lambda i: (0, i, 0))] * 5 +
                 [_const((HID, HID)), _const((1, HID)), _const((1, HID)),
                  _const((1, HID)), _const((HID, dff)), _const((1, dff)),
                  _const((dff, HID)), _const((1, HID)), _const((1, HID)),
                  _const((1, HID))],
        out_specs=_bs((BR, HID), lambda i: (i, 0)),
        out_shape=jax.ShapeDtypeStruct((N, HID), _f32),
    )(h, *parts, dgi, Wg, bg, g2, be2, W1, b1, W2, b2, g3, be3)


def _final_body(h, gf, bf, Wout, bout, o):
    y = _ln(h[...], gf[...], bf[...])
    o[...] = jnp.dot(y, Wout[...], preferred_element_type=_f32) + bout[...]


def _final(h, gf, bf, Wout, bout):
    dout = Wout.shape[1]
    return pl.pallas_call(
        _final_body, grid=(GR,),
        in_specs=[_bs((BR, HID), lambda i: (i, 0)), _const((1, HID)),
                  _const((1, HID)), _const((HID, dout)), _const((1, dout))],
        out_specs=_bs((BR, dout), lambda i: (i, 0)),
        out_shape=jax.ShapeDtypeStruct((N, dout), _f32),
    )(h, gf, bf, Wout, bout)


def _score_body(qs, ks, ebd, o):
    o[...] = jnp.dot(qs[...] * ks[...], ebd[...],
                     preferred_element_type=_f32) * (HD ** -0.5)


def _score_tc(gq, gk, ebd):
    return pl.pallas_call(
        _score_body, grid=(GE,),
        in_specs=[_bs((BE, HID), lambda i: (i, 0))] * 2 +
                 [_const((HID, HEADS))],
        out_specs=_bs((BE, HEADS), lambda i: (i, 0)),
        out_shape=jax.ShapeDtypeStruct((E, HEADS), _f32),
    )(gq, gk, ebd)


def _lp_body(a, b, w2p, b2s, o):
    r = jax.nn.relu(a[...] + b[...])
    o[...] = jax.nn.sigmoid(
        jnp.dot(r, w2p[...], preferred_element_type=_f32) + b2s[...])


def _lp_tc(ga, gb, w2p, b2s):
    return pl.pallas_call(
        _lp_body, grid=(GE,),
        in_specs=[_bs((BE, HID), lambda i: (i, 0))] * 2 +
                 [_const((HID, HEADS)), _const((1, HEADS))],
        out_specs=_bs((BE, HEADS), lambda i: (i, 0)),
        out_shape=jax.ShapeDtypeStruct((E, HEADS), _f32),
    )(ga, gb, w2p, b2s)


_INFO = plsc.get_sparse_core_info()
_NC = _INFO.num_cores
_NS = _INFO.num_subcores
_NW = _NC * _NS
EPW = E // _NW
CH = 40
NIT = EPW // CH
RB = 624
RTAIL = N - RB * _NS

_mesh = plsc.VectorSubcoreMesh(core_axis_name="c", subcore_axis_name="s")


def _make_gather(D):
    @functools.partial(
        pl.kernel, mesh=_mesh,
        out_type=jax.ShapeDtypeStruct((E, D), _f32),
        scratch_types=[pltpu.VMEM((NIT, CH), jnp.int32),
                       pltpu.VMEM((CH, D), _f32),
                       pltpu.SemaphoreType.DMA])
    def k(tab, idx3, out, idx_v, rows_v, sem):
        wid = lax.axis_index("s") * _NC + lax.axis_index("c")
        pltpu.sync_copy(idx3.at[wid], idx_v)

        def body(i, _):
            base = wid * EPW + i * CH
            pltpu.async_copy(tab.at[idx_v.at[i]], rows_v, sem).wait()
            pltpu.sync_copy(rows_v, out.at[pl.ds(base, CH)])
            return 0

        lax.fori_loop(0, NIT, body, 0)

    return k


def _make_scatter_rows(W):
    @functools.partial(
        pl.kernel, mesh=_mesh,
        out_type=jax.ShapeDtypeStruct((_NC, N, W), _f32),
        scratch_types=[pltpu.VMEM((CH,), jnp.int32),
                       pltpu.VMEM((CH, W), _f32),
                       pltpu.VMEM_SHARED((N, W), _f32)])
    def k(vals, idx, z, out, idx_v, vals_v, acc):
        cid = lax.axis_index("c")
        sid = lax.axis_index("s")
        wid = sid * _NC + cid
        r0 = sid * RB
        pltpu.sync_copy(z.at[pl.ds(r0, RB)], acc.at[pl.ds(r0, RB)])

        @pl.when(sid == _NS - 1)
        def _():
            pltpu.sync_copy(z.at[pl.ds(RB * _NS, RTAIL)],
                            acc.at[pl.ds(RB * _NS, RTAIL)])

        plsc.subcore_barrier()

        def body(i, _):
            base = wid * EPW + i * CH
            pltpu.sync_copy(idx.at[pl.ds(base, CH)], idx_v)
            pltpu.sync_copy(vals.at[pl.ds(base, CH)], vals_v)
            pltpu.sync_copy(vals_v, acc.at[idx_v], add=True)
            return 0

        lax.fori_loop(0, NIT, body, 0)
        plsc.subcore_barrier()
        pltpu.sync_copy(acc.at[pl.ds(r0, RB)], out.at[cid, pl.ds(r0, RB)])

        @pl.when(sid == _NS - 1)
        def _():
            pltpu.sync_copy(acc.at[pl.ds(RB * _NS, RTAIL)],
                            out.at[cid, pl.ds(RB * _NS, RTAIL)])

    return k


def _make_gss(S):
    @functools.partial(
        pl.kernel, mesh=_mesh,
        out_type=jax.ShapeDtypeStruct((_NC, N, CW), _f32),
        scratch_types=[pltpu.VMEM((CH,), jnp.int32),
                       pltpu.VMEM((CH,), jnp.int32),
                       pltpu.VMEM((CH * S + 16,), _f32),
                       pltpu.VMEM((CH, CW), _f32),
                       pltpu.VMEM_SHARED((N, CW), _f32),
                       pltpu.SemaphoreType.DMA])
    def k(tab, scl, gidx, sidx, z, out, gi_v, si_v, s_v, rows_v, acc, sem):
        cid = lax.axis_index("c")
        sid = lax.axis_index("s")
        wid = sid * _NC + cid
        r0 = sid * RB
        pltpu.sync_copy(z.at[pl.ds(r0, RB)], acc.at[pl.ds(r0, RB)])

        @pl.when(sid == _NS - 1)
        def _():
            pltpu.sync_copy(z.at[pl.ds(RB * _NS, RTAIL)],
                            acc.at[pl.ds(RB * _NS, RTAIL)])

        plsc.subcore_barrier()

        def body(i, _):
            base = wid * EPW + i * CH
            pltpu.sync_copy(gidx.at[pl.ds(base, CH)], gi_v)
            pltpu.sync_copy(sidx.at[pl.ds(base, CH)], si_v)
            pltpu.sync_copy(scl.at[pl.ds(base * S, CH * S)],
                            s_v.at[pl.ds(0, CH * S)])
            pltpu.async_copy(tab.at[gi_v], rows_v, sem).wait()

            def scale_edge(j, _):
                s0 = s_v[pl.ds(j * S, 16)][0]
                s1 = s0 if S == 1 else s_v[pl.ds(j * S + 1, 16)][0]
                for t in range(CW // 16):
                    s = s0 if t < (CW // 32) else s1
                    rows_v[j, pl.ds(t * 16, 16)] = (
                        rows_v[j, pl.ds(t * 16, 16)] * s)
                return 0

            lax.fori_loop(0, CH, scale_edge, 0)
            pltpu.sync_copy(rows_v, acc.at[si_v], add=True)
            return 0

        lax.fori_loop(0, NIT, body, 0)
        plsc.subcore_barrier()
        pltpu.sync_copy(acc.at[pl.ds(r0, RB)], out.at[cid, pl.ds(r0, RB)])

        @pl.when(sid == _NS - 1)
        def _():
            pltpu.sync_copy(acc.at[pl.ds(RB * _NS, RTAIL)],
                            out.at[cid, pl.ds(RB * _NS, RTAIL)])

    return k


def _make_gss_fused(S, with_extra):
    n_out = NCHUNK + (1 if with_extra else 0)
    scr = [pltpu.VMEM((NIT, CH), jnp.int32),
           pltpu.VMEM((NIT, CH), jnp.int32),
           pltpu.VMEM((CH * S + 16,), _f32),
           pltpu.VMEM((CH, CW), _f32),
           pltpu.VMEM_SHARED((N, CW), _f32),
           pltpu.SemaphoreType.DMA]

    @functools.partial(
        pl.kernel, mesh=_mesh,
        out_type=[jax.ShapeDtypeStruct((_NC, N, CW), _f32)] * n_out,
        scratch_types=scr)
    def k(*refs):
        tabs = refs[:NCHUNK]
        scl, gidx3, sidx3, z = refs[NCHUNK:NCHUNK + 4]
        pos = NCHUNK + 4
        extra = refs[pos] if with_extra else None
        pos += 1 if with_extra else 0
        outs = refs[pos:pos + n_out]
        gi_v, si_v, s_v, rows_v, acc, sem = refs[pos + n_out:]

        cid = lax.axis_index("c")
        sid = lax.axis_index("s")
        wid = sid * _NC + cid
        r0 = sid * RB
        pltpu.sync_copy(gidx3.at[wid], gi_v)
        pltpu.sync_copy(sidx3.at[wid], si_v)

        def zero_acc():
            pltpu.sync_copy(z.at[pl.ds(r0, RB)], acc.at[pl.ds(r0, RB)])

            @pl.when(sid == _NS - 1)
            def _():
                pltpu.sync_copy(z.at[pl.ds(RB * _NS, RTAIL)],
                                acc.at[pl.ds(RB * _NS, RTAIL)])

        def writeout(out):
            pltpu.sync_copy(acc.at[pl.ds(r0, RB)], out.at[cid, pl.ds(r0, RB)])

            @pl.when(sid == _NS - 1)
            def _():
                pltpu.sync_copy(acc.at[pl.ds(RB * _NS, RTAIL)],
                                out.at[cid, pl.ds(RB * _NS, RTAIL)])

        for c in range(NCHUNK):
            zero_acc()
            plsc.subcore_barrier()

            def body(i, _, c=c):
                base = wid * EPW + i * CH
                pltpu.sync_copy(scl.at[pl.ds(base * S, CH * S)],
                                s_v.at[pl.ds(0, CH * S)])
                pltpu.async_copy(tabs[c].at[gi_v.at[i]], rows_v, sem).wait()

                def scale_edge(j, _):
                    bs = j * S
                    s0 = s_v[pl.ds(bs + (2 * c if S > 1 else 0), 16)][0]
                    s1 = s0 if S == 1 else s_v[pl.ds(bs + 2 * c + 1, 16)][0]
                    for t in range(CW // 16):
                        s = s0 if t < (CW // 32) else s1
                        rows_v[j, pl.ds(t * 16, 16)] = (
                            rows_v[j, pl.ds(t * 16, 16)] * s)
                    return 0

                lax.fori_loop(0, CH, scale_edge, 0)
                pltpu.sync_copy(rows_v, acc.at[si_v.at[i]], add=True)
                return 0

            lax.fori_loop(0, NIT, body, 0)
            plsc.subcore_barrier()
            writeout(outs[c])
            plsc.subcore_barrier()

        if with_extra:
            zero_acc()
            plsc.subcore_barrier()

            def ebody(i, _):
                base = wid * EPW + i * CH
                pltpu.sync_copy(extra.at[pl.ds(base, CH)], rows_v)
                pltpu.sync_copy(rows_v, acc.at[si_v.at[i]], add=True)
                return 0

            lax.fori_loop(0, NIT, ebody, 0)
            plsc.subcore_barrier()
            writeout(outs[NCHUNK])

    return k


@functools.partial(
    pl.kernel, mesh=_mesh,
    out_type=[jax.ShapeDtypeStruct((_NC, N, CW), _f32)] * 2,
    scratch_types=[pltpu.VMEM((NIT, CH), jnp.int32),
                   pltpu.VMEM((NIT, CH), jnp.int32),
                   pltpu.VMEM((CH, CW), _f32),
                   pltpu.VMEM_SHARED((N, CW), _f32)])
def _deg_kernel(gidx3, sidx3, z, out_a, out_b, gi_v, si_v, rows_v, acc):
    cid = lax.axis_index("c")
    sid = lax.axis_index("s")
    wid = sid * _NC + cid
    r0 = sid * RB
    pltpu.sync_copy(gidx3.at[wid], gi_v)
    pltpu.sync_copy(sidx3.at[wid], si_v)

    def fill(j, _):
        for t in range(CW // 16):
            rows_v[j, pl.ds(t * 16, 16)] = jnp.ones((16,), _f32)
        return 0

    lax.fori_loop(0, CH, fill, 0)

    for idx_v, out in ((gi_v, out_a), (si_v, out_b)):
        pltpu.sync_copy(z.at[pl.ds(r0, RB)], acc.at[pl.ds(r0, RB)])

        @pl.when(sid == _NS - 1)
        def _():
            pltpu.sync_copy(z.at[pl.ds(RB * _NS, RTAIL)],
                            acc.at[pl.ds(RB * _NS, RTAIL)])

        plsc.subcore_barrier()

        def body(i, _, idx_v=idx_v):
            pltpu.sync_copy(rows_v, acc.at[idx_v.at[i]], add=True)
            return 0

        lax.fori_loop(0, NIT, body, 0)
        plsc.subcore_barrier()
        pltpu.sync_copy(acc.at[pl.ds(r0, RB)], out.at[cid, pl.ds(r0, RB)])

        @pl.when(sid == _NS - 1)
        def _():
            pltpu.sync_copy(acc.at[pl.ds(RB * _NS, RTAIL)],
                            out.at[cid, pl.ds(RB * _NS, RTAIL)])

        plsc.subcore_barrier()


_GATHER512 = _make_gather(HID)
_GSSF_ATTN = _make_gss_fused(HEADS, True)
_GSSF_GC = _make_gss_fused(1, False)


def kernel(node_features, edge_index, W_in, b_in, W_lp1, b_lp1, W_lp2, b_lp2,
           Wq, bq, Wk, bk, Wv, bv, Wo, bo, Wg, bg, W1, b1, W2, b2,
           g1, be1, g2, be2, g3, be3, gf, bf, W_out, b_out):
    src = edge_index[0]
    dst = edge_index[1]
    r2 = lambda v: v.reshape(1, -1)

    src3 = src.reshape(_NW, NIT, CH)
    dst3 = dst.reshape(_NW, NIT, CH)

    h, A2, B2 = _t0(node_features, W_in, r2(b_in), W_lp1[:HID], W_lp1[HID:],
                    r2(b_lp1))
    ga = _GATHER512(A2, src3)
    gb = _GATHER512(B2, dst3)
    w2p = jnp.tile(W_lp2, (1, HEADS))
    b2s = jnp.broadcast_to(b_lp2.reshape(1, 1), (1, HEADS))
    ews = _lp_tc(ga, gb, w2p, b2s)[:, 0]

    z128 = jnp.zeros((N, CW), _f32)
    epad = jnp.zeros((E, CW - HEADS), _f32)
    dgo, dgi = _deg_kernel(src3, dst3, z128)

    erep = jnp.repeat(jnp.eye(HEADS, dtype=_f32), HD, axis=1)
    ebd = jnp.repeat(jnp.eye(HEADS, dtype=_f32), HD, axis=0)

    for l in range(3):
        q, k, v = _qkv(h, Wq[l], r2(bq[l]), Wk[l], r2(bk[l]), Wv[l],
                       r2(bv[l]))
        gq = _GATHER512(q, src3)
        gk = _GATHER512(k, dst3)
        sc = _score_tc(gq, gk, ebd)
        e = _expk(sc, _gmax(sc))
        vtabs = [v[:, c * CW:(c + 1) * CW] for c in range(NCHUNK)]
        *parts, dnm = _GSSF_ATTN(*vtabs, e.reshape(-1), src3, dst3, z128,
                                 jnp.concatenate([e, epad], axis=1))
        h, feat = _attn(h, parts, dnm, dgo, erep, Wo[l], r2(bo[l]),
                        r2(g1[l]), r2(be1[l]))
        ftabs = [feat[:, c * CW:(c + 1) * CW] for c in range(NCHUNK)]
        gparts = _GSSF_GC(*ftabs, ews, src3, dst3, z128)
        h = _gcff(h, gparts, dgi, Wg[l], r2(bg[l]), r2(g2[l]), r2(be2[l]),
                  W1[l], r2(b1[l]), W2[l], r2(b2[l]), r2(g3[l]), r2(be3[l]))

    return _final(h, r2(gf), r2(bf), W_out, r2(b_out))


# speedup vs baseline: 8.6280x; 1.3449x over previous
import functools

import jax
import jax.numpy as jnp
from jax import lax
from jax.experimental import pallas as pl
from jax.experimental.pallas import tpu as pltpu
from jax.experimental.pallas import tpu_sc as plsc

N = 10000
E = 160000
HID = 512
HEADS = 8
HD = HID // HEADS
NCHUNK = 4
CW = HID // NCHUNK

BR = 400
GR = N // BR
BE = 2000
GE = E // BE

_f32 = jnp.float32


def _bs(shape, imap):
    return pl.BlockSpec(shape, imap)


def _const(shape):
    return pl.BlockSpec(shape, lambda i: (0,) * len(shape))


def _t0_body(x, Win, bin_, Wt, Wb, blp, h_o, a_o, b_o):
    h = jnp.dot(x[...], Win[...], preferred_element_type=_f32) + bin_[...]
    h_o[...] = h
    a_o[...] = jnp.dot(h, Wt[...], preferred_element_type=_f32) + blp[...]
    b_o[...] = jnp.dot(h, Wb[...], preferred_element_type=_f32)


def _t0(x, Win, bin_, Wt, Wb, blp):
    d_in = x.shape[1]
    return pl.pallas_call(
        _t0_body, grid=(GR,),
        in_specs=[_bs((BR, d_in), lambda i: (i, 0)), _const((d_in, HID)),
                  _const((1, HID)), _const((HID, HID)), _const((HID, HID)),
                  _const((1, HID))],
        out_specs=[_bs((BR, HID), lambda i: (i, 0))] * 3,
        out_shape=[jax.ShapeDtypeStruct((N, HID), _f32)] * 3,
    )(x, Win, bin_, Wt, Wb, blp)


def _qkv_body(h, Wq, bq, Wk, bk, Wv, bv, q_o, k_o, v_o):
    hh = h[...]
    q_o[...] = jnp.dot(hh, Wq[...], preferred_element_type=_f32) + bq[...]
    k_o[...] = jnp.dot(hh, Wk[...], preferred_element_type=_f32) + bk[...]
    v_o[...] = jnp.dot(hh, Wv[...], preferred_element_type=_f32) + bv[...]


def _qkv(h, Wq, bq, Wk, bk, Wv, bv):
    return pl.pallas_call(
        _qkv_body, grid=(GR,),
        in_specs=[_bs((BR, HID), lambda i: (i, 0))] +
                 [_const((HID, HID)), _const((1, HID))] * 3,
        out_specs=[_bs((BR, HID), lambda i: (i, 0))] * 3,
        out_shape=[jax.ShapeDtypeStruct((N, HID), _f32)] * 3,
    )(h, Wq, bq, Wk, bk, Wv, bv)


def _gmax_body(sc, o):
    i = pl.program_id(0)

    @pl.when(i == 0)
    def _():
        o[...] = jnp.full_like(o[...], -jnp.inf)

    o[...] = jnp.maximum(o[...], jnp.max(sc[...], axis=0, keepdims=True))


def _gmax(sc):
    return pl.pallas_call(
        _gmax_body, grid=(GE,),
        in_specs=[_bs((BE, HEADS), lambda i: (i, 0))],
        out_specs=_bs((1, HEADS), lambda i: (0, 0)),
        out_shape=jax.ShapeDtypeStruct((1, HEADS), _f32),
    )(sc)


def _exp_body(sc, gm, o):
    o[...] = jnp.exp(sc[...] - gm[...])


def _expk(sc, gm):
    return pl.pallas_call(
        _exp_body, grid=(GE,),
        in_specs=[_bs((BE, HEADS), lambda i: (i, 0)), _const((1, HEADS))],
        out_specs=_bs((BE, HEADS), lambda i: (i, 0)),
        out_shape=jax.ShapeDtypeStruct((E, HEADS), _f32),
    )(sc, gm)


def _ln(x, g, b, eps=1e-5):
    mu = jnp.mean(x, axis=-1, keepdims=True)
    var = jnp.mean((x - mu) ** 2, axis=-1, keepdims=True)
    return (x - mu) * lax.rsqrt(var + eps) * g + b


def _attn_body(h, u0, u1, u2, u3, dnm, dgo, erep, Wo, bo, g1, be1,
               h_o, f_o):
    u = jnp.concatenate(
        [u0[0] + u0[1], u1[0] + u1[1], u2[0] + u2[1], u3[0] + u3[1]], axis=1)
    den = dnm[0][:, :HEADS] + dnm[1][:, :HEADS]
    denx = jnp.dot(den, erep[...], preferred_element_type=_f32)
    hag = u / jnp.maximum(denx, 1e-30)
    h_attn = jnp.dot(hag, Wo[...], preferred_element_type=_f32) + bo[...]
    y = _ln(h[...] + h_attn, g1[...], be1[...])
    h_o[...] = y
    deg = jnp.maximum(dgo[0][:, :1] + dgo[1][:, :1], 1.0)
    f_o[...] = y * lax.rsqrt(deg)


def _attn(h, parts, dnm, dgo, erep, Wo, bo, g1, be1):
    return pl.pallas_call(
        _attn_body, grid=(GR,),
        in_specs=[_bs((BR, HID), lambda i: (i, 0))] +
                 [_bs((2, BR, CW), lambda i: (0, i, 0))] * 6 +
                 [_const((HEADS, HID)), _const((HID, HID)), _const((1, HID)),
                  _const((1, HID)), _const((1, HID))],
        out_specs=[_bs((BR, HID), lambda i: (i, 0))] * 2,
        out_shape=[jax.ShapeDtypeStruct((N, HID), _f32)] * 2,
    )(h, *parts, dnm, dgo, erep, Wo, bo, g1, be1)


def _gcff_body(h, a0, a1, a2, a3, dgi, Wg, bg, g2, be2, W1, b1, W2, b2,
               g3, be3, h_o):
    agg = jnp.concatenate(
        [a0[0] + a0[1], a1[0] + a1[1], a2[0] + a2[1], a3[0] + a3[1]], axis=1)
    deg = jnp.maximum(dgi[0][:, :1] + dgi[1][:, :1], 1.0)
    agg = agg * lax.rsqrt(deg)
    hs = jnp.dot(agg, Wg[...], preferred_element_type=_f32) + bg[...]
    y = _ln(h[...] + hs, g2[...], be2[...])
    f = jnp.dot(jax.nn.relu(
        jnp.dot(y, W1[...], preferred_element_type=_f32) + b1[...]),
        W2[...], preferred_element_type=_f32) + b2[...]
    h_o[...] = _ln(y + f, g3[...], be3[...])


def _gcff(h, parts, dgi, Wg, bg, g2, be2, W1, b1, W2, b2, g3, be3):
    dff = W1.shape[1]
    return pl.pallas_call(
        _gcff_body, grid=(GR,),
        in_specs=[_bs((BR, HID), lambda i: (i, 0))] +
                 [_bs((2, BR, CW), lambda i: (0, i, 0))] * 5 +
                 [_const((HID, HID)), _const((1, HID)), _const((1, HID)),
                  _const((1, HID)), _const((HID, dff)), _const((1, dff)),
                  _const((dff, HID)), _const((1, HID)), _const((1, HID)),
                  _const((1, HID))],
        out_specs=_bs((BR, HID), lambda i: (i, 0)),
        out_shape=jax.ShapeDtypeStruct((N, HID), _f32),
    )(h, *parts, dgi, Wg, bg, g2, be2, W1, b1, W2, b2, g3, be3)


def _final_body(h, gf, bf, Wout, bout, o):
    y = _ln(h[...], gf[...], bf[...])
    o[...] = jnp.dot(y, Wout[...], preferred_element_type=_f32) + bout[...]


def _final(h, gf, bf, Wout, bout):
    dout = Wout.shape[1]
    return pl.pallas_call(
        _final_body, grid=(GR,),
        in_specs=[_bs((BR, HID), lambda i: (i, 0)), _const((1, HID)),
                  _const((1, HID)), _const((HID, dout)), _const((1, dout))],
        out_specs=_bs((BR, dout), lambda i: (i, 0)),
        out_shape=jax.ShapeDtypeStruct((N, dout), _f32),
    )(h, gf, bf, Wout, bout)


def _score_body(qs, ks, ebd, o):
    o[...] = jnp.dot(qs[...] * ks[...], ebd[...],
                     preferred_element_type=_f32) * (HD ** -0.5)


def _score_tc(gq, gk, ebd):
    return pl.pallas_call(
        _score_body, grid=(GE,),
        in_specs=[_bs((BE, HID), lambda i: (i, 0))] * 2 +
                 [_const((HID, HEADS))],
        out_specs=_bs((BE, HEADS), lambda i: (i, 0)),
        out_shape=jax.ShapeDtypeStruct((E, HEADS), _f32),
    )(gq, gk, ebd)


def _lp_body(a, b, w2p, b2s, o):
    r = jax.nn.relu(a[...] + b[...])
    o[...] = jax.nn.sigmoid(
        jnp.dot(r, w2p[...], preferred_element_type=_f32) + b2s[...])


def _lp_tc(ga, gb, w2p, b2s):
    return pl.pallas_call(
        _lp_body, grid=(GE,),
        in_specs=[_bs((BE, HID), lambda i: (i, 0))] * 2 +
                 [_const((HID, HEADS)), _const((1, HEADS))],
        out_specs=_bs((BE, HEADS), lambda i: (i, 0)),
        out_shape=jax.ShapeDtypeStruct((E, HEADS), _f32),
    )(ga, gb, w2p, b2s)


_INFO = plsc.get_sparse_core_info()
_NC = _INFO.num_cores
_NS = _INFO.num_subcores
_NW = _NC * _NS
EPW = E // _NW
CH = 40
NIT = EPW // CH
RB = 624
RTAIL = N - RB * _NS

_mesh = plsc.VectorSubcoreMesh(core_axis_name="c", subcore_axis_name="s")


def _make_gather(D):
    @functools.partial(
        pl.kernel, mesh=_mesh,
        out_type=jax.ShapeDtypeStruct((E, D), _f32),
        scratch_types=[pltpu.VMEM((NIT, CH), jnp.int32),
                       pltpu.VMEM((CH, D), _f32),
                       pltpu.VMEM((CH, D), _f32),
                       pltpu.SemaphoreType.DMA,
                       pltpu.SemaphoreType.DMA])
    def k(tab, idx3, out, idx_v, rows_a, rows_b, sem_a, sem_b):
        wid = lax.axis_index("s") * _NC + lax.axis_index("c")
        pltpu.sync_copy(idx3.at[wid], idx_v)
        pltpu.async_copy(tab.at[idx_v.at[0]], rows_a, sem_a)

        def pair(t, _):
            i0 = 2 * t
            pltpu.make_async_copy(tab.at[idx_v.at[i0]], rows_a, sem_a).wait()
            pltpu.async_copy(tab.at[idx_v.at[i0 + 1]], rows_b, sem_b)
            pltpu.sync_copy(rows_a, out.at[pl.ds(wid * EPW + i0 * CH, CH)])
            pltpu.make_async_copy(tab.at[idx_v.at[i0 + 1]], rows_b,
                                  sem_b).wait()
            pltpu.async_copy(tab.at[idx_v.at[i0 + 2]], rows_a, sem_a)
            pltpu.sync_copy(rows_b,
                            out.at[pl.ds(wid * EPW + (i0 + 1) * CH, CH)])
            return 0

        lax.fori_loop(0, NIT // 2, pair, 0)
        pltpu.make_async_copy(tab.at[idx_v.at[NIT - 1]], rows_a, sem_a).wait()
        pltpu.sync_copy(rows_a, out.at[pl.ds(wid * EPW + (NIT - 1) * CH, CH)])

    return k


def _make_scatter_rows(W):
    @functools.partial(
        pl.kernel, mesh=_mesh,
        out_type=jax.ShapeDtypeStruct((_NC, N, W), _f32),
        scratch_types=[pltpu.VMEM((CH,), jnp.int32),
                       pltpu.VMEM((CH, W), _f32),
                       pltpu.VMEM_SHARED((N, W), _f32)])
    def k(vals, idx, z, out, idx_v, vals_v, acc):
        cid = lax.axis_index("c")
        sid = lax.axis_index("s")
        wid = sid * _NC + cid
        r0 = sid * RB
        pltpu.sync_copy(z.at[pl.ds(r0, RB)], acc.at[pl.ds(r0, RB)])

        @pl.when(sid == _NS - 1)
        def _():
            pltpu.sync_copy(z.at[pl.ds(RB * _NS, RTAIL)],
                            acc.at[pl.ds(RB * _NS, RTAIL)])

        plsc.subcore_barrier()

        def body(i, _):
            base = wid * EPW + i * CH
            pltpu.sync_copy(idx.at[pl.ds(base, CH)], idx_v)
            pltpu.sync_copy(vals.at[pl.ds(base, CH)], vals_v)
            pltpu.sync_copy(vals_v, acc.at[idx_v], add=True)
            return 0

        lax.fori_loop(0, NIT, body, 0)
        plsc.subcore_barrier()
        pltpu.sync_copy(acc.at[pl.ds(r0, RB)], out.at[cid, pl.ds(r0, RB)])

        @pl.when(sid == _NS - 1)
        def _():
            pltpu.sync_copy(acc.at[pl.ds(RB * _NS, RTAIL)],
                            out.at[cid, pl.ds(RB * _NS, RTAIL)])

    return k


def _make_gss(S):
    @functools.partial(
        pl.kernel, mesh=_mesh,
        out_type=jax.ShapeDtypeStruct((_NC, N, CW), _f32),
        scratch_types=[pltpu.VMEM((CH,), jnp.int32),
                       pltpu.VMEM((CH,), jnp.int32),
                       pltpu.VMEM((CH * S + 16,), _f32),
                       pltpu.VMEM((CH, CW), _f32),
                       pltpu.VMEM_SHARED((N, CW), _f32),
                       pltpu.SemaphoreType.DMA])
    def k(tab, scl, gidx, sidx, z, out, gi_v, si_v, s_v, rows_v, acc, sem):
        cid = lax.axis_index("c")
        sid = lax.axis_index("s")
        wid = sid * _NC + cid
        r0 = sid * RB
        pltpu.sync_copy(z.at[pl.ds(r0, RB)], acc.at[pl.ds(r0, RB)])

        @pl.when(sid == _NS - 1)
        def _():
            pltpu.sync_copy(z.at[pl.ds(RB * _NS, RTAIL)],
                            acc.at[pl.ds(RB * _NS, RTAIL)])

        plsc.subcore_barrier()

        def body(i, _):
            base = wid * EPW + i * CH
            pltpu.sync_copy(gidx.at[pl.ds(base, CH)], gi_v)
            pltpu.sync_copy(sidx.at[pl.ds(base, CH)], si_v)
            pltpu.sync_copy(scl.at[pl.ds(base * S, CH * S)],
                            s_v.at[pl.ds(0, CH * S)])
            pltpu.async_copy(tab.at[gi_v], rows_v, sem).wait()

            def scale_edge(j, _):
                s0 = s_v[pl.ds(j * S, 16)][0]
                s1 = s0 if S == 1 else s_v[pl.ds(j * S + 1, 16)][0]
                for t in range(CW // 16):
                    s = s0 if t < (CW // 32) else s1
                    rows_v[j, pl.ds(t * 16, 16)] = (
                        rows_v[j, pl.ds(t * 16, 16)] * s)
                return 0

            lax.fori_loop(0, CH, scale_edge, 0)
            pltpu.sync_copy(rows_v, acc.at[si_v], add=True)
            return 0

        lax.fori_loop(0, NIT, body, 0)
        plsc.subcore_barrier()
        pltpu.sync_copy(acc.at[pl.ds(r0, RB)], out.at[cid, pl.ds(r0, RB)])

        @pl.when(sid == _NS - 1)
        def _():
            pltpu.sync_copy(acc.at[pl.ds(RB * _NS, RTAIL)],
                            out.at[cid, pl.ds(RB * _NS, RTAIL)])

    return k


def _make_gss_fused(S, with_extra):
    n_out = NCHUNK + (1 if with_extra else 0)
    scr = [pltpu.VMEM((NIT, CH), jnp.int32),
           pltpu.VMEM((NIT, CH), jnp.int32),
           pltpu.VMEM((CH * S + 16,), _f32),
           pltpu.VMEM((CH, CW), _f32),
           pltpu.VMEM((CH, CW), _f32),
           pltpu.VMEM_SHARED((N, CW), _f32),
           pltpu.SemaphoreType.DMA,
           pltpu.SemaphoreType.DMA]

    @functools.partial(
        pl.kernel, mesh=_mesh,
        out_type=[jax.ShapeDtypeStruct((_NC, N, CW), _f32)] * n_out,
        scratch_types=scr)
    def k(*refs):
        tabs = refs[:NCHUNK]
        scl, gidx3, sidx3, z = refs[NCHUNK:NCHUNK + 4]
        pos = NCHUNK + 4
        extra = refs[pos] if with_extra else None
        pos += 1 if with_extra else 0
        outs = refs[pos:pos + n_out]
        (gi_v, si_v, s_v, rows_a, rows_b, acc, sem_a,
         sem_b) = refs[pos + n_out:]

        cid = lax.axis_index("c")
        sid = lax.axis_index("s")
        wid = sid * _NC + cid
        r0 = sid * RB
        pltpu.sync_copy(gidx3.at[wid], gi_v)
        pltpu.sync_copy(sidx3.at[wid], si_v)

        def zero_acc():
            pltpu.sync_copy(z.at[pl.ds(r0, RB)], acc.at[pl.ds(r0, RB)])

            @pl.when(sid == _NS - 1)
            def _():
                pltpu.sync_copy(z.at[pl.ds(RB * _NS, RTAIL)],
                                acc.at[pl.ds(RB * _NS, RTAIL)])

        def writeout(out):
            pltpu.sync_copy(acc.at[pl.ds(r0, RB)], out.at[cid, pl.ds(r0, RB)])

            @pl.when(sid == _NS - 1)
            def _():
                pltpu.sync_copy(acc.at[pl.ds(RB * _NS, RTAIL)],
                                out.at[cid, pl.ds(RB * _NS, RTAIL)])

        for c in range(NCHUNK):
            pltpu.async_copy(tabs[c].at[gi_v.at[0]], rows_a, sem_a)
            zero_acc()
            plsc.subcore_barrier()

            def scale_scatter(i, rows_v, c=c):
                base = wid * EPW + i * CH
                pltpu.sync_copy(scl.at[pl.ds(base * S, CH * S)],
                                s_v.at[pl.ds(0, CH * S)])

                def scale_edge(j, _):
                    bs = j * S
                    s0 = s_v[pl.ds(bs + (2 * c if S > 1 else 0), 16)][0]
                    s1 = s0 if S == 1 else s_v[pl.ds(bs + 2 * c + 1, 16)][0]
                    for t in range(CW // 16):
                        s = s0 if t < (CW // 32) else s1
                        rows_v[j, pl.ds(t * 16, 16)] = (
                            rows_v[j, pl.ds(t * 16, 16)] * s)
                    return 0

                lax.fori_loop(0, CH, scale_edge, 0)
                pltpu.sync_copy(rows_v, acc.at[si_v.at[i]], add=True)

            def pair(t, _, c=c):
                i0 = 2 * t
                pltpu.make_async_copy(tabs[c].at[gi_v.at[i0]], rows_a,
                                      sem_a).wait()
                pltpu.async_copy(tabs[c].at[gi_v.at[i0 + 1]], rows_b, sem_b)
                scale_scatter(i0, rows_a)
                pltpu.make_async_copy(tabs[c].at[gi_v.at[i0 + 1]], rows_b,
                                      sem_b).wait()
                pltpu.async_copy(tabs[c].at[gi_v.at[i0 + 2]], rows_a, sem_a)
                scale_scatter(i0 + 1, rows_b)
                return 0

            lax.fori_loop(0, NIT // 2, pair, 0)
            pltpu.make_async_copy(tabs[c].at[gi_v.at[NIT - 1]], rows_a,
                                  sem_a).wait()
            scale_scatter(NIT - 1, rows_a)
            plsc.subcore_barrier()
            writeout(outs[c])
            plsc.subcore_barrier()

        if with_extra:
            zero_acc()
            plsc.subcore_barrier()

            def ebody(i, _):
                base = wid * EPW + i * CH
                pltpu.sync_copy(extra.at[pl.ds(base, CH)], rows_a)
                pltpu.sync_copy(rows_a, acc.at[si_v.at[i]], add=True)
                return 0

            lax.fori_loop(0, NIT, ebody, 0)
            plsc.subcore_barrier()
            writeout(outs[NCHUNK])

    return k


@functools.partial(
    pl.kernel, mesh=_mesh,
    out_type=[jax.ShapeDtypeStruct((_NC, N, CW), _f32)] * 2,
    scratch_types=[pltpu.VMEM((NIT, CH), jnp.int32),
                   pltpu.VMEM((NIT, CH), jnp.int32),
                   pltpu.VMEM((CH, CW), _f32),
                   pltpu.VMEM_SHARED((N, CW), _f32)])
def _deg_kernel(gidx3, sidx3, z, out_a, out_b, gi_v, si_v, rows_v, acc):
    cid = lax.axis_index("c")
    sid = lax.axis_index("s")
    wid = sid * _NC + cid
    r0 = sid * RB
    pltpu.sync_copy(gidx3.at[wid], gi_v)
    pltpu.sync_copy(sidx3.at[wid], si_v)

    def fill(j, _):
        for t in range(CW // 16):
            rows_v[j, pl.ds(t * 16, 16)] = jnp.ones((16,), _f32)
        return 0

    lax.fori_loop(0, CH, fill, 0)

    for idx_v, out in ((gi_v, out_a), (si_v, out_b)):
        pltpu.sync_copy(z.at[pl.ds(r0, RB)], acc.at[pl.ds(r0, RB)])

        @pl.when(sid == _NS - 1)
        def _():
            pltpu.sync_copy(z.at[pl.ds(RB * _NS, RTAIL)],
                            acc.at[pl.ds(RB * _NS, RTAIL)])

        plsc.subcore_barrier()

        def body(i, _, idx_v=idx_v):
            pltpu.sync_copy(rows_v, acc.at[idx_v.at[i]], add=True)
            return 0

        lax.fori_loop(0, NIT, body, 0)
        plsc.subcore_barrier()
        pltpu.sync_copy(acc.at[pl.ds(r0, RB)], out.at[cid, pl.ds(r0, RB)])

        @pl.when(sid == _NS - 1)
        def _():
            pltpu.sync_copy(acc.at[pl.ds(RB * _NS, RTAIL)],
                            out.at[cid, pl.ds(RB * _NS, RTAIL)])

        plsc.subcore_barrier()


_GATHER512 = _make_gather(HID)
_GSSF_ATTN = _make_gss_fused(HEADS, True)
_GSSF_GC = _make_gss_fused(1, False)


def kernel(node_features, edge_index, W_in, b_in, W_lp1, b_lp1, W_lp2, b_lp2,
           Wq, bq, Wk, bk, Wv, bv, Wo, bo, Wg, bg, W1, b1, W2, b2,
           g1, be1, g2, be2, g3, be3, gf, bf, W_out, b_out):
    src = edge_index[0]
    dst = edge_index[1]
    r2 = lambda v: v.reshape(1, -1)

    src3 = src.reshape(_NW, NIT, CH)
    dst3 = dst.reshape(_NW, NIT, CH)

    h, A2, B2 = _t0(node_features, W_in, r2(b_in), W_lp1[:HID], W_lp1[HID:],
                    r2(b_lp1))
    ga = _GATHER512(A2, src3)
    gb = _GATHER512(B2, dst3)
    w2p = jnp.tile(W_lp2, (1, HEADS))
    b2s = jnp.broadcast_to(b_lp2.reshape(1, 1), (1, HEADS))
    ews = _lp_tc(ga, gb, w2p, b2s)[:, 0]

    z128 = jnp.zeros((N, CW), _f32)
    epad = jnp.zeros((E, CW - HEADS), _f32)
    dgo, dgi = _deg_kernel(src3, dst3, z128)

    erep = jnp.repeat(jnp.eye(HEADS, dtype=_f32), HD, axis=1)
    ebd = jnp.repeat(jnp.eye(HEADS, dtype=_f32), HD, axis=0)

    for l in range(3):
        q, k, v = _qkv(h, Wq[l], r2(bq[l]), Wk[l], r2(bk[l]), Wv[l],
                       r2(bv[l]))
        gq = _GATHER512(q, src3)
        gk = _GATHER512(k, dst3)
        sc = _score_tc(gq, gk, ebd)
        e = _expk(sc, _gmax(sc))
        vtabs = [v[:, c * CW:(c + 1) * CW] for c in range(NCHUNK)]
        *parts, dnm = _GSSF_ATTN(*vtabs, e.reshape(-1), src3, dst3, z128,
                                 jnp.concatenate([e, epad], axis=1))
        h, feat = _attn(h, parts, dnm, dgo, erep, Wo[l], r2(bo[l]),
                        r2(g1[l]), r2(be1[l]))
        ftabs = [feat[:, c * CW:(c + 1) * CW] for c in range(NCHUNK)]
        gparts = _GSSF_GC(*ftabs, ews, src3, dst3, z128)
        h = _gcff(h, gparts, dgi, Wg[l], r2(bg[l]), r2(g2[l]), r2(be2[l]),
                  W1[l], r2(b1[l]), W2[l], r2(b2[l]), r2(g3[l]), r2(be3[l]))

    return _final(h, r2(gf), r2(bf), W_out, r2(b_out))


# speedup vs baseline: 8.6334x; 1.0006x over previous
import functools

import jax
import jax.numpy as jnp
from jax import lax
from jax.experimental import pallas as pl
from jax.experimental.pallas import tpu as pltpu
from jax.experimental.pallas import tpu_sc as plsc

N = 10000
E = 160000
HID = 512
HEADS = 8
HD = HID // HEADS
NCHUNK = 4
CW = HID // NCHUNK

BR = 400
GR = N // BR
BE = 2000
GE = E // BE

_f32 = jnp.float32


def _bs(shape, imap):
    return pl.BlockSpec(shape, imap)


def _const(shape):
    return pl.BlockSpec(shape, lambda i: (0,) * len(shape))


def _t0_body(x, Win, bin_, Wt, Wb, blp, h_o, a_o, b_o):
    h = jnp.dot(x[...], Win[...], preferred_element_type=_f32) + bin_[...]
    h_o[...] = h
    a_o[...] = jnp.dot(h, Wt[...], preferred_element_type=_f32) + blp[...]
    b_o[...] = jnp.dot(h, Wb[...], preferred_element_type=_f32)


def _t0(x, Win, bin_, Wt, Wb, blp):
    d_in = x.shape[1]
    return pl.pallas_call(
        _t0_body, grid=(GR,),
        in_specs=[_bs((BR, d_in), lambda i: (i, 0)), _const((d_in, HID)),
                  _const((1, HID)), _const((HID, HID)), _const((HID, HID)),
                  _const((1, HID))],
        out_specs=[_bs((BR, HID), lambda i: (i, 0))] * 3,
        out_shape=[jax.ShapeDtypeStruct((N, HID), _f32)] * 3,
    )(x, Win, bin_, Wt, Wb, blp)


def _qkv_body(h, Wq, bq, Wk, bk, Wv, bv, q_o, k_o, v_o):
    hh = h[...]
    q_o[...] = jnp.dot(hh, Wq[...], preferred_element_type=_f32) + bq[...]
    k_o[...] = jnp.dot(hh, Wk[...], preferred_element_type=_f32) + bk[...]
    v_o[...] = jnp.dot(hh, Wv[...], preferred_element_type=_f32) + bv[...]


def _qkv(h, Wq, bq, Wk, bk, Wv, bv):
    return pl.pallas_call(
        _qkv_body, grid=(GR,),
        in_specs=[_bs((BR, HID), lambda i: (i, 0))] +
                 [_const((HID, HID)), _const((1, HID))] * 3,
        out_specs=[_bs((BR, HID), lambda i: (i, 0))] * 3,
        out_shape=[jax.ShapeDtypeStruct((N, HID), _f32)] * 3,
    )(h, Wq, bq, Wk, bk, Wv, bv)


def _gmax_body(sc, o):
    i = pl.program_id(0)

    @pl.when(i == 0)
    def _():
        o[...] = jnp.full_like(o[...], -jnp.inf)

    o[...] = jnp.maximum(o[...], jnp.max(sc[...], axis=0, keepdims=True))


def _gmax(sc):
    return pl.pallas_call(
        _gmax_body, grid=(GE,),
        in_specs=[_bs((BE, HEADS), lambda i: (i, 0))],
        out_specs=_bs((1, HEADS), lambda i: (0, 0)),
        out_shape=jax.ShapeDtypeStruct((1, HEADS), _f32),
    )(sc)


def _exp_body(sc, gm, o):
    o[...] = jnp.exp(sc[...] - gm[...])


def _expk(sc, gm):
    return pl.pallas_call(
        _exp_body, grid=(GE,),
        in_specs=[_bs((BE, HEADS), lambda i: (i, 0)), _const((1, HEADS))],
        out_specs=_bs((BE, HEADS), lambda i: (i, 0)),
        out_shape=jax.ShapeDtypeStruct((E, HEADS), _f32),
    )(sc, gm)


def _ln(x, g, b, eps=1e-5):
    mu = jnp.mean(x, axis=-1, keepdims=True)
    var = jnp.mean((x - mu) ** 2, axis=-1, keepdims=True)
    return (x - mu) * lax.rsqrt(var + eps) * g + b


def _attn_body(h, u0, u1, u2, u3, dnm, dgo, erep, Wo, bo, g1, be1,
               h_o, f_o):
    u = jnp.concatenate(
        [u0[0] + u0[1], u1[0] + u1[1], u2[0] + u2[1], u3[0] + u3[1]], axis=1)
    den = dnm[0][:, :HEADS] + dnm[1][:, :HEADS]
    denx = jnp.dot(den, erep[...], preferred_element_type=_f32)
    hag = u / jnp.maximum(denx, 1e-30)
    h_attn = jnp.dot(hag, Wo[...], preferred_element_type=_f32) + bo[...]
    y = _ln(h[...] + h_attn, g1[...], be1[...])
    h_o[...] = y
    deg = jnp.maximum(dgo[0][:, :1] + dgo[1][:, :1], 1.0)
    f_o[...] = y * lax.rsqrt(deg)


def _attn(h, parts, dnm, dgo, erep, Wo, bo, g1, be1):
    return pl.pallas_call(
        _attn_body, grid=(GR,),
        in_specs=[_bs((BR, HID), lambda i: (i, 0))] +
                 [_bs((2, BR, CW), lambda i: (0, i, 0))] * 6 +
                 [_const((HEADS, HID)), _const((HID, HID)), _const((1, HID)),
                  _const((1, HID)), _const((1, HID))],
        out_specs=[_bs((BR, HID), lambda i: (i, 0))] * 2,
        out_shape=[jax.ShapeDtypeStruct((N, HID), _f32)] * 2,
    )(h, *parts, dnm, dgo, erep, Wo, bo, g1, be1)


def _gcff_body(h, a0, a1, a2, a3, dgi, Wg, bg, g2, be2, W1, b1, W2, b2,
               g3, be3, h_o):
    agg = jnp.concatenate(
        [a0[0] + a0[1], a1[0] + a1[1], a2[0] + a2[1], a3[0] + a3[1]], axis=1)
    deg = jnp.maximum(dgi[0][:, :1] + dgi[1][:, :1], 1.0)
    agg = agg * lax.rsqrt(deg)
    hs = jnp.dot(agg, Wg[...], preferred_element_type=_f32) + bg[...]
    y = _ln(h[...] + hs, g2[...], be2[...])
    f = jnp.dot(jax.nn.relu(
        jnp.dot(y, W1[...], preferred_element_type=_f32) + b1[...]),
        W2[...], preferred_element_type=_f32) + b2[...]
    h_o[...] = _ln(y + f, g3[...], be3[...])


def _gcff(h, parts, dgi, Wg, bg, g2, be2, W1, b1, W2, b2, g3, be3):
    dff = W1.shape[1]
    return pl.pallas_call(
        _gcff_body, grid=(GR,),
        in_specs=[_bs((BR, HID), lambda i: (i, 0))] +
                 [_bs((2, BR, CW), lambda i: (0, i, 0))] * 5 +
                 [_const((HID, HID)), _const((1, HID)), _const((1, HID)),
                  _const((1, HID)), _const((HID, dff)), _const((1, dff)),
                  _const((dff, HID)), _const((1, HID)), _const((1, HID)),
                  _const((1, HID))],
        out_specs=_bs((BR, HID), lambda i: (i, 0)),
        out_shape=jax.ShapeDtypeStruct((N, HID), _f32),
    )(h, *parts, dgi, Wg, bg, g2, be2, W1, b1, W2, b2, g3, be3)


def _final_body(h, gf, bf, Wout, bout, o):
    y = _ln(h[...], gf[...], bf[...])
    o[...] = jnp.dot(y, Wout[...], preferred_element_type=_f32) + bout[...]


def _final(h, gf, bf, Wout, bout):
    dout = Wout.shape[1]
    return pl.pallas_call(
        _final_body, grid=(GR,),
        in_specs=[_bs((BR, HID), lambda i: (i, 0)), _const((1, HID)),
                  _const((1, HID)), _const((HID, dout)), _const((1, dout))],
        out_specs=_bs((BR, dout), lambda i: (i, 0)),
        out_shape=jax.ShapeDtypeStruct((N, dout), _f32),
    )(h, gf, bf, Wout, bout)


def _score_body(qs, ks, ebd, o):
    o[...] = jnp.dot(qs[...] * ks[...], ebd[...],
                     preferred_element_type=_f32) * (HD ** -0.5)


def _score_tc(gq, gk, ebd):
    return pl.pallas_call(
        _score_body, grid=(GE,),
        in_specs=[_bs((BE, HID), lambda i: (i, 0))] * 2 +
                 [_const((HID, HEADS))],
        out_specs=_bs((BE, HEADS), lambda i: (i, 0)),
        out_shape=jax.ShapeDtypeStruct((E, HEADS), _f32),
    )(gq, gk, ebd)


def _lp_body(a, b, w2p, b2s, o):
    r = jax.nn.relu(a[...] + b[...])
    o[...] = jax.nn.sigmoid(
        jnp.dot(r, w2p[...], preferred_element_type=_f32) + b2s[...])


def _lp_tc(ga, gb, w2p, b2s):
    return pl.pallas_call(
        _lp_body, grid=(GE,),
        in_specs=[_bs((BE, HID), lambda i: (i, 0))] * 2 +
                 [_const((HID, HEADS)), _const((1, HEADS))],
        out_specs=_bs((BE, HEADS), lambda i: (i, 0)),
        out_shape=jax.ShapeDtypeStruct((E, HEADS), _f32),
    )(ga, gb, w2p, b2s)


_INFO = plsc.get_sparse_core_info()
_NC = _INFO.num_cores
_NS = _INFO.num_subcores
_NW = _NC * _NS
EPW = E // _NW
CH = 40
NIT = EPW // CH
RB = 624
RTAIL = N - RB * _NS

_mesh = plsc.VectorSubcoreMesh(core_axis_name="c", subcore_axis_name="s")


def _make_gather(D):
    @functools.partial(
        pl.kernel, mesh=_mesh,
        out_type=jax.ShapeDtypeStruct((E, D), _f32),
        scratch_types=[pltpu.VMEM((NIT, CH), jnp.int32),
                       pltpu.VMEM((CH, D), _f32),
                       pltpu.VMEM((CH, D), _f32),
                       pltpu.SemaphoreType.DMA,
                       pltpu.SemaphoreType.DMA])
    def k(tab, idx3, out, idx_v, rows_a, rows_b, sem_a, sem_b):
        wid = lax.axis_index("s") * _NC + lax.axis_index("c")
        pltpu.sync_copy(idx3.at[wid], idx_v)
        pltpu.async_copy(tab.at[idx_v.at[0]], rows_a, sem_a)

        def pair(t, _):
            i0 = 2 * t
            pltpu.make_async_copy(tab.at[idx_v.at[i0]], rows_a, sem_a).wait()
            pltpu.async_copy(tab.at[idx_v.at[i0 + 1]], rows_b, sem_b)
            pltpu.sync_copy(rows_a, out.at[pl.ds(wid * EPW + i0 * CH, CH)])
            pltpu.make_async_copy(tab.at[idx_v.at[i0 + 1]], rows_b,
                                  sem_b).wait()
            pltpu.async_copy(tab.at[idx_v.at[i0 + 2]], rows_a, sem_a)
            pltpu.sync_copy(rows_b,
                            out.at[pl.ds(wid * EPW + (i0 + 1) * CH, CH)])
            return 0

        lax.fori_loop(0, NIT // 2, pair, 0)
        pltpu.make_async_copy(tab.at[idx_v.at[NIT - 1]], rows_a, sem_a).wait()
        pltpu.sync_copy(rows_a, out.at[pl.ds(wid * EPW + (NIT - 1) * CH, CH)])

    return k


def _make_gss_fused(S, with_extra):
    n_out = NCHUNK + (1 if with_extra else 0)
    scr = [pltpu.VMEM((NIT, CH), jnp.int32),
           pltpu.VMEM((NIT, CH), jnp.int32),
           pltpu.VMEM((CH * S + 16,), _f32),
           pltpu.VMEM((CH, CW), _f32),
           pltpu.VMEM((CH, CW), _f32),
           pltpu.VMEM_SHARED((N, CW), _f32),
           pltpu.SemaphoreType.DMA,
           pltpu.SemaphoreType.DMA]

    @functools.partial(
        pl.kernel, mesh=_mesh,
        out_type=[jax.ShapeDtypeStruct((_NC, N, CW), _f32)] * n_out,
        scratch_types=scr)
    def k(*refs):
        tabs = refs[:NCHUNK]
        scl, gidx3, sidx3, z = refs[NCHUNK:NCHUNK + 4]
        pos = NCHUNK + 4
        extra = refs[pos] if with_extra else None
        pos += 1 if with_extra else 0
        outs = refs[pos:pos + n_out]
        (gi_v, si_v, s_v, rows_a, rows_b, acc, sem_a,
         sem_b) = refs[pos + n_out:]

        cid = lax.axis_index("c")
        sid = lax.axis_index("s")
        wid = sid * _NC + cid
        r0 = sid * RB
        pltpu.sync_copy(gidx3.at[wid], gi_v)
        pltpu.sync_copy(sidx3.at[wid], si_v)

        def zero_acc():
            pltpu.sync_copy(z.at[pl.ds(r0, RB)], acc.at[pl.ds(r0, RB)])

            @pl.when(sid == _NS - 1)
            def _():
                pltpu.sync_copy(z.at[pl.ds(RB * _NS, RTAIL)],
                                acc.at[pl.ds(RB * _NS, RTAIL)])

        def writeout(out):
            pltpu.sync_copy(acc.at[pl.ds(r0, RB)], out.at[cid, pl.ds(r0, RB)])

            @pl.when(sid == _NS - 1)
            def _():
                pltpu.sync_copy(acc.at[pl.ds(RB * _NS, RTAIL)],
                                out.at[cid, pl.ds(RB * _NS, RTAIL)])

        for c in range(NCHUNK):
            pltpu.async_copy(tabs[c].at[gi_v.at[0]], rows_a, sem_a)
            zero_acc()
            plsc.subcore_barrier()

            def scale_scatter(i, rows_v, c=c):
                base = wid * EPW + i * CH
                pltpu.sync_copy(scl.at[pl.ds(base * S, CH * S)],
                                s_v.at[pl.ds(0, CH * S)])

                def scale_edge(j, _):
                    bs = j * S
                    s0 = s_v[pl.ds(bs + (2 * c if S > 1 else 0), 16)][0]
                    s1 = s0 if S == 1 else s_v[pl.ds(bs + 2 * c + 1, 16)][0]
                    for t in range(CW // 16):
                        s = s0 if t < (CW // 32) else s1
                        rows_v[j, pl.ds(t * 16, 16)] = (
                            rows_v[j, pl.ds(t * 16, 16)] * s)
                    return 0

                lax.fori_loop(0, CH, scale_edge, 0)
                pltpu.sync_copy(rows_v, acc.at[si_v.at[i]], add=True)

            def pair(t, _, c=c):
                i0 = 2 * t
                pltpu.make_async_copy(tabs[c].at[gi_v.at[i0]], rows_a,
                                      sem_a).wait()
                pltpu.async_copy(tabs[c].at[gi_v.at[i0 + 1]], rows_b, sem_b)
                scale_scatter(i0, rows_a)
                pltpu.make_async_copy(tabs[c].at[gi_v.at[i0 + 1]], rows_b,
                                      sem_b).wait()
                pltpu.async_copy(tabs[c].at[gi_v.at[i0 + 2]], rows_a, sem_a)
                scale_scatter(i0 + 1, rows_b)
                return 0

            lax.fori_loop(0, NIT // 2, pair, 0)
            pltpu.make_async_copy(tabs[c].at[gi_v.at[NIT - 1]], rows_a,
                                  sem_a).wait()
            scale_scatter(NIT - 1, rows_a)
            plsc.subcore_barrier()
            writeout(outs[c])
            plsc.subcore_barrier()

        if with_extra:
            zero_acc()
            plsc.subcore_barrier()

            def ebody(i, _):
                base = wid * EPW + i * CH
                pltpu.sync_copy(extra.at[pl.ds(base, CH)], rows_a)
                pltpu.sync_copy(rows_a, acc.at[si_v.at[i]], add=True)
                return 0

            lax.fori_loop(0, NIT, ebody, 0)
            plsc.subcore_barrier()
            writeout(outs[NCHUNK])

    return k


@functools.partial(
    pl.kernel, mesh=_mesh,
    out_type=[jax.ShapeDtypeStruct((_NC, N, CW), _f32)] * 2,
    scratch_types=[pltpu.VMEM((NIT, CH), jnp.int32),
                   pltpu.VMEM((NIT, CH), jnp.int32),
                   pltpu.VMEM((CH, CW), _f32),
                   pltpu.VMEM_SHARED((N, CW), _f32)])
def _deg_kernel(gidx3, sidx3, z, out_a, out_b, gi_v, si_v, rows_v, acc):
    cid = lax.axis_index("c")
    sid = lax.axis_index("s")
    wid = sid * _NC + cid
    r0 = sid * RB
    pltpu.sync_copy(gidx3.at[wid], gi_v)
    pltpu.sync_copy(sidx3.at[wid], si_v)

    def fill(j, _):
        for t in range(CW // 16):
            rows_v[j, pl.ds(t * 16, 16)] = jnp.ones((16,), _f32)
        return 0

    lax.fori_loop(0, CH, fill, 0)

    for idx_v, out in ((gi_v, out_a), (si_v, out_b)):
        pltpu.sync_copy(z.at[pl.ds(r0, RB)], acc.at[pl.ds(r0, RB)])

        @pl.when(sid == _NS - 1)
        def _():
            pltpu.sync_copy(z.at[pl.ds(RB * _NS, RTAIL)],
                            acc.at[pl.ds(RB * _NS, RTAIL)])

        plsc.subcore_barrier()

        def body(i, _, idx_v=idx_v):
            pltpu.sync_copy(rows_v, acc.at[idx_v.at[i]], add=True)
            return 0

        lax.fori_loop(0, NIT, body, 0)
        plsc.subcore_barrier()
        pltpu.sync_copy(acc.at[pl.ds(r0, RB)], out.at[cid, pl.ds(r0, RB)])

        @pl.when(sid == _NS - 1)
        def _():
            pltpu.sync_copy(acc.at[pl.ds(RB * _NS, RTAIL)],
                            out.at[cid, pl.ds(RB * _NS, RTAIL)])

        plsc.subcore_barrier()


_GATHER512 = _make_gather(HID)
_GSSF_ATTN = _make_gss_fused(HEADS, True)
_GSSF_GC = _make_gss_fused(1, False)


def kernel(node_features, edge_index, W_in, b_in, W_lp1, b_lp1, W_lp2, b_lp2,
           Wq, bq, Wk, bk, Wv, bv, Wo, bo, Wg, bg, W1, b1, W2, b2,
           g1, be1, g2, be2, g3, be3, gf, bf, W_out, b_out):
    src = edge_index[0]
    dst = edge_index[1]
    r2 = lambda v: v.reshape(1, -1)

    src3 = src.reshape(_NW, NIT, CH)
    dst3 = dst.reshape(_NW, NIT, CH)

    h, A2, B2 = _t0(node_features, W_in, r2(b_in), W_lp1[:HID], W_lp1[HID:],
                    r2(b_lp1))
    ga = _GATHER512(A2, src3)
    gb = _GATHER512(B2, dst3)
    w2p = jnp.tile(W_lp2, (1, HEADS))
    b2s = jnp.broadcast_to(b_lp2.reshape(1, 1), (1, HEADS))
    ews = _lp_tc(ga, gb, w2p, b2s)[:, 0]

    z128 = jnp.zeros((N, CW), _f32)
    epad = jnp.zeros((E, CW - HEADS), _f32)
    dgo, dgi = _deg_kernel(src3, dst3, z128)

    erep = jnp.repeat(jnp.eye(HEADS, dtype=_f32), HD, axis=1)
    ebd = jnp.repeat(jnp.eye(HEADS, dtype=_f32), HD, axis=0)

    for l in range(3):
        q, k, v = _qkv(h, Wq[l], r2(bq[l]), Wk[l], r2(bk[l]), Wv[l],
                       r2(bv[l]))
        gq = _GATHER512(q, src3)
        gk = _GATHER512(k, dst3)
        sc = _score_tc(gq, gk, ebd)
        e = _expk(sc, _gmax(sc))
        vtabs = [v[:, c * CW:(c + 1) * CW] for c in range(NCHUNK)]
        *parts, dnm = _GSSF_ATTN(*vtabs, e.reshape(-1), src3, dst3, z128,
                                 jnp.concatenate([e, epad], axis=1))
        h, feat = _attn(h, parts, dnm, dgo, erep, Wo[l], r2(bo[l]),
                        r2(g1[l]), r2(be1[l]))
        ftabs = [feat[:, c * CW:(c + 1) * CW] for c in range(NCHUNK)]
        gparts = _GSSF_GC(*ftabs, ews, src3, dst3, z128)
        h = _gcff(h, gparts, dgi, Wg[l], r2(bg[l]), r2(g2[l]), r2(be2[l]),
                  W1[l], r2(b1[l]), W2[l], r2(b2[l]), r2(g3[l]), r2(be3[l]))

    return _final(h, r2(gf), r2(bf), W_out, r2(b_out))
